# baseline (device time: 164095 ns/iter reference)
import jax
import jax.numpy as jnp
from jax import lax
from jax.experimental import pallas as pl
from jax.experimental.pallas import tpu as pltpu

N_DEV = 4
SUB = 64
NSUB = 4
GEMM_ROWS = 128
HOPS = N_DEV - 1


def kernel(x, w_mat):
    m, k = x.shape
    _, n = w_mat.shape

    def body(x_ref, w_ref, out_ref, comm_ref, send_sems, recv_sems):
        my = lax.axis_index("i")
        left = lax.rem(my + N_DEV - 1, N_DEV)
        right = lax.rem(my + 1, N_DEV)
        nbr = (right, left)

        barrier_sem = pltpu.get_barrier_semaphore()
        for b in (left, right):
            pl.semaphore_signal(
                barrier_sem, inc=1,
                device_id=(b,), device_id_type=pl.DeviceIdType.MESH,
            )
        pl.semaphore_wait(barrier_sem, 2)

        def gemm(c):
            r = pl.ds(c * GEMM_ROWS, GEMM_ROWS)
            out_ref[r, :] = jnp.dot(
                x_ref[r, :], w_ref[...], preferred_element_type=jnp.float32,
            )

        def rows(j, d, u):
            return pl.ds((8 * j + 4 * d + u) * SUB, SUB)

        def rs_j(d, s, send):
            off = s if send else s + 1
            if d == 0:
                return lax.rem(my - off + N_DEV, N_DEV)
            return lax.rem(my + off, N_DEV)

        def ag_j(d, g):
            if d == 0:
                return lax.rem(my + 1 - g + N_DEV, N_DEV)
            return lax.rem(my - 1 + g + N_DEV, N_DEV)

        all_rdmas = []

        def rs_rdma(d, s, u):
            i = NSUB * s + u
            r = pltpu.make_async_remote_copy(
                src_ref=out_ref.at[rows(rs_j(d, s, True), d, u), :],
                dst_ref=comm_ref.at[d, i],
                send_sem=send_sems.at[d, i],
                recv_sem=recv_sems.at[d, i],
                device_id=(nbr[d],),
                device_id_type=pl.DeviceIdType.MESH,
            )
            all_rdmas.append(r)
            return r

        def ag_rdma(d, g, u):
            i = NSUB * (HOPS + g) + u
            sl = rows(ag_j(d, g), d, u)
            r = pltpu.make_async_remote_copy(
                src_ref=out_ref.at[sl, :],
                dst_ref=out_ref.at[sl, :],
                send_sem=send_sems.at[d, i],
                recv_sem=recv_sems.at[d, i],
                device_id=(nbr[d],),
                device_id_type=pl.DeviceIdType.MESH,
            )
            all_rdmas.append(r)
            return r

        live = {}

        def start(key, r):
            live[key] = r
            r.start()

        for t in range(4):
            gemm(4 * my + t)
            d = t // 2
            for u in (2 * (t % 2), 2 * (t % 2) + 1):
                start((d, 0, u), rs_rdma(d, 0, u))

        for q in (1, 2, 3):
            for t in range(4):
                gemm(4 * lax.rem(my + q, N_DEV) + t)

        for s in range(HOPS):
            for u in range(NSUB):
                for d in (0, 1):
                    live[(d, s, u)].wait_recv()
                    dst = rows(rs_j(d, s, False), d, u)
                    out_ref[dst, :] = (
                        out_ref[dst, :] + comm_ref[d, NSUB * s + u]
                    )
                    if s + 1 < HOPS:
                        start((d, s + 1, u), rs_rdma(d, s + 1, u))
                    else:
                        start((d, HOPS, u), ag_rdma(d, 0, u))

        for g in range(HOPS):
            for u in range(NSUB):
                for d in (0, 1):
                    live[(d, HOPS + g, u)].wait_recv()
                    if g + 1 < HOPS:
                        start((d, HOPS + g + 1, u), ag_rdma(d, g + 1, u))

        for r in all_rdmas:
            r.wait_send()

    return pl.pallas_call(
        body,
        out_shape=jax.ShapeDtypeStruct((m, n), jnp.float32),
        in_specs=[
            pl.BlockSpec(memory_space=pltpu.VMEM),
            pl.BlockSpec(memory_space=pltpu.VMEM),
        ],
        out_specs=pl.BlockSpec(memory_space=pltpu.VMEM),
        scratch_shapes=[
            pltpu.VMEM((2, NSUB * HOPS, SUB, n), jnp.float32),
            pltpu.SemaphoreType.DMA((2, NSUB * 2 * HOPS)),
            pltpu.SemaphoreType.DMA((2, NSUB * 2 * HOPS)),
        ],
        compiler_params=pltpu.CompilerParams(
            collective_id=0, vmem_limit_bytes=100 * 1024 * 1024,
        ),
    )(x, w_mat)


# device time: 97074 ns/iter; 1.6904x vs baseline; 1.6904x over previous
import jax
import jax.numpy as jnp
from jax import lax
from jax.experimental import pallas as pl
from jax.experimental.pallas import tpu as pltpu

N_DEV = 4
SUB = 64
NSUB = 4
GEMM_ROWS = 128
HOPS = N_DEV - 1


def kernel(x, w_mat):
    m, k = x.shape
    _, n = w_mat.shape

    def body(
        x_ref, w_ref, out_ref,
        rs_stage, rs_comm, ag_stage, ag_comm,
        rs_send_sems, rs_recv_sems, ag_send_sems, ag_recv_sems,
    ):
        my = lax.axis_index("i")
        left = lax.rem(my + N_DEV - 1, N_DEV)
        right = lax.rem(my + 1, N_DEV)
        nbr = (right, left)

        barrier_sem = pltpu.get_barrier_semaphore()
        for b in (left, right):
            pl.semaphore_signal(
                barrier_sem, inc=1,
                device_id=(b,), device_id_type=pl.DeviceIdType.MESH,
            )
        pl.semaphore_wait(barrier_sem, 2)

        def gemm(c):
            r = pl.ds(c * GEMM_ROWS, GEMM_ROWS)
            out_ref[r, :] = jnp.dot(
                x_ref[r, :], w_ref[...], preferred_element_type=jnp.float32,
            )

        def rows(j, d, u):
            return pl.ds((8 * j + 4 * d + u) * SUB, SUB)

        def rs_j(d, s, send):
            off = s if send else s + 1
            if d == 0:
                return lax.rem(my - off + N_DEV, N_DEV)
            return lax.rem(my + off, N_DEV)

        def ag_recv_j(d, g):
            if d == 0:
                return lax.rem(my - g + N_DEV, N_DEV)
            return lax.rem(my + g, N_DEV)

        all_rdmas = []
        live = {}

        def start(key, r):
            all_rdmas.append(r)
            live[key] = r
            r.start()

        def rs_start(d, s, u):
            i = NSUB * s + u
            rs_stage[d, i] = out_ref[rows(rs_j(d, s, True), d, u), :].astype(
                jnp.bfloat16
            )
            start((d, s, u), pltpu.make_async_remote_copy(
                src_ref=rs_stage.at[d, i],
                dst_ref=rs_comm.at[d, i],
                send_sem=rs_send_sems.at[d, i],
                recv_sem=rs_recv_sems.at[d, i],
                device_id=(nbr[d],),
                device_id_type=pl.DeviceIdType.MESH,
            ))

        def ag_start(d, g, u):
            i = NSUB * g + u
            src = ag_stage.at[d, u] if g == 0 else ag_comm.at[d, NSUB * (g - 1) + u]
            start((d, HOPS + g, u), pltpu.make_async_remote_copy(
                src_ref=src,
                dst_ref=ag_comm.at[d, i],
                send_sem=ag_send_sems.at[d, i],
                recv_sem=ag_recv_sems.at[d, i],
                device_id=(nbr[d],),
                device_id_type=pl.DeviceIdType.MESH,
            ))

        for t in range(4):
            gemm(4 * my + t)
            d = t // 2
            for u in (2 * (t % 2), 2 * (t % 2) + 1):
                rs_start(d, 0, u)

        for q in (1, 2, 3):
            for t in range(4):
                gemm(4 * lax.rem(my + q, N_DEV) + t)

        for s in range(HOPS):
            for u in range(NSUB):
                for d in (0, 1):
                    live[(d, s, u)].wait_recv()
                    dst = rows(rs_j(d, s, False), d, u)
                    out_ref[dst, :] = (
                        out_ref[dst, :]
                        + rs_comm[d, NSUB * s + u].astype(jnp.float32)
                    )
                    if s + 1 < HOPS:
                        rs_start(d, s + 1, u)
                    else:
                        ag_stage[d, u] = out_ref[dst, :].astype(jnp.bfloat16)
                        ag_start(d, 0, u)

        for g in range(HOPS):
            for u in range(NSUB):
                for d in (0, 1):
                    live[(d, HOPS + g, u)].wait_recv()
                    sl = rows(ag_recv_j(d, g), d, u)
                    out_ref[sl, :] = ag_comm[d, NSUB * g + u].astype(jnp.float32)
                    if g + 1 < HOPS:
                        ag_start(d, g + 1, u)

        for r in all_rdmas:
            r.wait_send()

    nsl = NSUB * HOPS
    return pl.pallas_call(
        body,
        out_shape=jax.ShapeDtypeStruct((m, n), jnp.float32),
        in_specs=[
            pl.BlockSpec(memory_space=pltpu.VMEM),
            pl.BlockSpec(memory_space=pltpu.VMEM),
        ],
        out_specs=pl.BlockSpec(memory_space=pltpu.VMEM),
        scratch_shapes=[
            pltpu.VMEM((2, nsl, SUB, n), jnp.bfloat16),
            pltpu.VMEM((2, nsl, SUB, n), jnp.bfloat16),
            pltpu.VMEM((2, NSUB, SUB, n), jnp.bfloat16),
            pltpu.VMEM((2, nsl, SUB, n), jnp.bfloat16),
            pltpu.SemaphoreType.DMA((2, nsl)),
            pltpu.SemaphoreType.DMA((2, nsl)),
            pltpu.SemaphoreType.DMA((2, nsl)),
            pltpu.SemaphoreType.DMA((2, nsl)),
        ],
        compiler_params=pltpu.CompilerParams(
            collective_id=0, vmem_limit_bytes=100 * 1024 * 1024,
        ),
    )(x, w_mat)


# device time: 97010 ns/iter; 1.6915x vs baseline; 1.0007x over previous
import jax
import jax.numpy as jnp
from jax import lax
from jax.experimental import pallas as pl
from jax.experimental.pallas import tpu as pltpu

N_DEV = 4
SUB = 64
NSUB = 4
GEMM_ROWS = 128
HOPS = N_DEV - 1


def kernel(x, w_mat):
    m, k = x.shape
    _, n = w_mat.shape

    def body(
        x_ref, w_ref, out_ref,
        rs_stage, rs_comm, ag_stage, ag_comm,
        rs_send_sems, rs_recv_sems, ag_send_sems, ag_recv_sems,
    ):
        my = lax.axis_index("i")
        left = lax.rem(my + N_DEV - 1, N_DEV)
        right = lax.rem(my + 1, N_DEV)
        nbr = (right, left)

        barrier_sem = pltpu.get_barrier_semaphore()
        for b in (left, right):
            pl.semaphore_signal(
                barrier_sem, inc=1,
                device_id=(b,), device_id_type=pl.DeviceIdType.MESH,
            )
        pl.semaphore_wait(barrier_sem, 2)

        def gemm(c):
            r = pl.ds(c * GEMM_ROWS, GEMM_ROWS)
            out_ref[r, :] = jnp.dot(
                x_ref[r, :], w_ref[...], preferred_element_type=jnp.float32,
            )

        def rows(j, d, u):
            return pl.ds((8 * j + 4 * d + u) * SUB, SUB)

        def rs_j(d, s, send):
            off = s if send else s + 1
            if d == 0:
                return lax.rem(my - off + N_DEV, N_DEV)
            return lax.rem(my + off, N_DEV)

        def ag_recv_j(d, g):
            if d == 0:
                return lax.rem(my - g + N_DEV, N_DEV)
            return lax.rem(my + g, N_DEV)

        all_rdmas = []
        live = {}

        def start(key, r):
            all_rdmas.append(r)
            live[key] = r
            r.start()

        def rs_start(d, s, u, val=None):
            i = NSUB * s + u
            if val is None:
                val = out_ref[rows(rs_j(d, s, True), d, u), :]
            rs_stage[d, i] = val.astype(jnp.bfloat16)
            start((d, s, u), pltpu.make_async_remote_copy(
                src_ref=rs_stage.at[d, i],
                dst_ref=rs_comm.at[d, i],
                send_sem=rs_send_sems.at[d, i],
                recv_sem=rs_recv_sems.at[d, i],
                device_id=(nbr[d],),
                device_id_type=pl.DeviceIdType.MESH,
            ))

        def ag_start(d, g, u):
            i = NSUB * g + u
            src = ag_stage.at[d, u] if g == 0 else ag_comm.at[d, NSUB * (g - 1) + u]
            start((d, HOPS + g, u), pltpu.make_async_remote_copy(
                src_ref=src,
                dst_ref=ag_comm.at[d, i],
                send_sem=ag_send_sems.at[d, i],
                recv_sem=ag_recv_sems.at[d, i],
                device_id=(nbr[d],),
                device_id_type=pl.DeviceIdType.MESH,
            ))

        for t in range(4):
            gemm(4 * my + t)
            d = t // 2
            for u in (2 * (t % 2), 2 * (t % 2) + 1):
                rs_start(d, 0, u)

        def rs_process(d, s, u):
            live[(d, s, u)].wait_recv()
            dst = rows(rs_j(d, s, False), d, u)
            tmp = out_ref[dst, :] + rs_comm[d, NSUB * s + u].astype(jnp.float32)
            out_ref[dst, :] = tmp
            if s + 1 < HOPS:
                rs_start(d, s + 1, u, val=tmp)
            else:
                ag_stage[d, u] = tmp.astype(jnp.bfloat16)
                ag_start(d, 0, u)

        for q in (1, 3):
            for t in range(4):
                gemm(4 * lax.rem(my + q, N_DEV) + t)

        for u in range(NSUB):
            for d in (0, 1):
                rs_process(d, 0, u)

        for t in range(4):
            gemm(4 * lax.rem(my + 2, N_DEV) + t)

        for s in range(1, HOPS):
            for u in range(NSUB):
                for d in (0, 1):
                    rs_process(d, s, u)

        for g in range(HOPS):
            for u in range(NSUB):
                for d in (0, 1):
                    live[(d, HOPS + g, u)].wait_recv()
                    sl = rows(ag_recv_j(d, g), d, u)
                    out_ref[sl, :] = ag_comm[d, NSUB * g + u].astype(jnp.float32)
                    if g + 1 < HOPS:
                        ag_start(d, g + 1, u)

        for r in all_rdmas:
            r.wait_send()

    nsl = NSUB * HOPS
    return pl.pallas_call(
        body,
        out_shape=jax.ShapeDtypeStruct((m, n), jnp.float32),
        in_specs=[
            pl.BlockSpec(memory_space=pltpu.VMEM),
            pl.BlockSpec(memory_space=pltpu.VMEM),
        ],
        out_specs=pl.BlockSpec(memory_space=pltpu.VMEM),
        scratch_shapes=[
            pltpu.VMEM((2, nsl, SUB, n), jnp.bfloat16),
            pltpu.VMEM((2, nsl, SUB, n), jnp.bfloat16),
            pltpu.VMEM((2, NSUB, SUB, n), jnp.bfloat16),
            pltpu.VMEM((2, nsl, SUB, n), jnp.bfloat16),
            pltpu.SemaphoreType.DMA((2, nsl)),
            pltpu.SemaphoreType.DMA((2, nsl)),
            pltpu.SemaphoreType.DMA((2, nsl)),
            pltpu.SemaphoreType.DMA((2, nsl)),
        ],
        compiler_params=pltpu.CompilerParams(
            collective_id=0, vmem_limit_bytes=100 * 1024 * 1024,
        ),
    )(x, w_mat)


# device time: 92358 ns/iter; 1.7767x vs baseline; 1.0504x over previous
import jax
import jax.numpy as jnp
from jax import lax
from jax.experimental import pallas as pl
from jax.experimental.pallas import tpu as pltpu

N_DEV = 4
SUB = 64
NSUB = 4
GEMM_ROWS = 128
HOPS = N_DEV - 1


def kernel(x, w_mat):
    m, k = x.shape
    _, n = w_mat.shape

    def body(
        x_ref, w_ref, out_ref,
        acc, rs_stage, rs_comm, ag_stage, ag_comm,
        rs_send_sems, rs_recv_sems, ag_send_sems, ag_recv_sems, out_sems,
    ):
        my = lax.axis_index("i")
        left = lax.rem(my + N_DEV - 1, N_DEV)
        right = lax.rem(my + 1, N_DEV)
        nbr = (right, left)

        barrier_sem = pltpu.get_barrier_semaphore()
        for b in (left, right):
            pl.semaphore_signal(
                barrier_sem, inc=1,
                device_id=(b,), device_id_type=pl.DeviceIdType.MESH,
            )
        pl.semaphore_wait(barrier_sem, 2)

        out_copies = []

        def flush(sl, d, i):
            cp = pltpu.make_async_copy(acc.at[sl, :], out_ref.at[sl, :],
                                       out_sems.at[d, i])
            cp.start()
            out_copies.append(cp)

        def gemm(c):
            r = pl.ds(c * GEMM_ROWS, GEMM_ROWS)
            acc[r, :] = jnp.dot(
                x_ref[r, :], w_ref[...], preferred_element_type=jnp.float32,
            )

        def rows(j, d, u):
            return pl.ds((8 * j + 4 * d + u) * SUB, SUB)

        def rs_j(d, s, send):
            off = s if send else s + 1
            if d == 0:
                return lax.rem(my - off + N_DEV, N_DEV)
            return lax.rem(my + off, N_DEV)

        def ag_recv_j(d, g):
            if d == 0:
                return lax.rem(my - g + N_DEV, N_DEV)
            return lax.rem(my + g, N_DEV)

        all_rdmas = []
        live = {}

        def start(key, r):
            all_rdmas.append(r)
            live[key] = r
            r.start()

        def rs_start(d, s, u, val=None):
            i = NSUB * s + u
            if val is None:
                val = acc[rows(rs_j(d, s, True), d, u), :]
            rs_stage[d, i] = val.astype(jnp.bfloat16)
            start((d, s, u), pltpu.make_async_remote_copy(
                src_ref=rs_stage.at[d, i],
                dst_ref=rs_comm.at[d, i],
                send_sem=rs_send_sems.at[d, i],
                recv_sem=rs_recv_sems.at[d, i],
                device_id=(nbr[d],),
                device_id_type=pl.DeviceIdType.MESH,
            ))

        def ag_start(d, g, u):
            i = NSUB * g + u
            src = ag_stage.at[d, u] if g == 0 else ag_comm.at[d, NSUB * (g - 1) + u]
            start((d, HOPS + g, u), pltpu.make_async_remote_copy(
                src_ref=src,
                dst_ref=ag_comm.at[d, i],
                send_sem=ag_send_sems.at[d, i],
                recv_sem=ag_recv_sems.at[d, i],
                device_id=(nbr[d],),
                device_id_type=pl.DeviceIdType.MESH,
            ))

        for t in range(4):
            gemm(4 * my + t)
            d = t // 2
            for u in (2 * (t % 2), 2 * (t % 2) + 1):
                rs_start(d, 0, u)

        def rs_process(d, s, u):
            live[(d, s, u)].wait_recv()
            dst = rows(rs_j(d, s, False), d, u)
            tmp = acc[dst, :] + rs_comm[d, NSUB * s + u].astype(jnp.float32)
            acc[dst, :] = tmp
            if s + 1 < HOPS:
                rs_start(d, s + 1, u, val=tmp)
            else:
                ag_stage[d, u] = tmp.astype(jnp.bfloat16)
                ag_start(d, 0, u)
                flush(dst, d, u)

        for q in (1, 3):
            for t in range(4):
                gemm(4 * lax.rem(my + q, N_DEV) + t)

        for u in range(NSUB):
            for d in (0, 1):
                rs_process(d, 0, u)

        for t in range(4):
            gemm(4 * lax.rem(my + 2, N_DEV) + t)

        for s in range(1, HOPS):
            for u in range(NSUB):
                for d in (0, 1):
                    rs_process(d, s, u)

        for g in range(HOPS):
            for u in range(NSUB):
                for d in (0, 1):
                    live[(d, HOPS + g, u)].wait_recv()
                    sl = rows(ag_recv_j(d, g), d, u)
                    acc[sl, :] = ag_comm[d, NSUB * g + u].astype(jnp.float32)
                    if g + 1 < HOPS:
                        ag_start(d, g + 1, u)
                    flush(sl, d, NSUB + NSUB * g + u)

        for cp in out_copies:
            cp.wait()
        for r in all_rdmas:
            r.wait_send()

    nsl = NSUB * HOPS
    return pl.pallas_call(
        body,
        out_shape=jax.ShapeDtypeStruct((m, n), jnp.float32),
        in_specs=[
            pl.BlockSpec(memory_space=pltpu.VMEM),
            pl.BlockSpec(memory_space=pltpu.VMEM),
        ],
        out_specs=pl.BlockSpec(memory_space=pl.ANY),
        scratch_shapes=[
            pltpu.VMEM((m, n), jnp.float32),
            pltpu.VMEM((2, nsl, SUB, n), jnp.bfloat16),
            pltpu.VMEM((2, nsl, SUB, n), jnp.bfloat16),
            pltpu.VMEM((2, NSUB, SUB, n), jnp.bfloat16),
            pltpu.VMEM((2, nsl, SUB, n), jnp.bfloat16),
            pltpu.SemaphoreType.DMA((2, nsl)),
            pltpu.SemaphoreType.DMA((2, nsl)),
            pltpu.SemaphoreType.DMA((2, nsl)),
            pltpu.SemaphoreType.DMA((2, nsl)),
            pltpu.SemaphoreType.DMA((2, NSUB + nsl)),
        ],
        compiler_params=pltpu.CompilerParams(
            collective_id=0, vmem_limit_bytes=100 * 1024 * 1024,
        ),
    )(x, w_mat)
